# Initial kernel scaffold; baseline (speedup 1.0000x reference)
import jax
import jax.numpy as jnp
from jax.experimental import pallas as pl


def kernel(question_emb, entity_table, relation_table, W_ent, W_q, W_edge, node_embedding_ids, node_global_ids, edge_index, edge_relations, edge_batch):
    raise NotImplementedError("write your pallas kernel here")


# speedup vs baseline: 3.4439x; 3.4439x over previous
import functools

import jax
import jax.numpy as jnp
from jax import lax
from jax.experimental import pallas as pl
from jax.experimental.pallas import tpu as pltpu
from jax.experimental.pallas import tpu_sc as plsc

D = 128
NC = 2
NS = 16
NW = NC * NS
CH = 128


def _round_up(x, m):
    return (x + m - 1) // m * m


def _entity_gather_body(tab_hbm, idx_hbm, out_hbm, idx_v, rows_v, sem):
    w = lax.axis_index("s") * NC + lax.axis_index("c")
    n_pad = out_hbm.shape[0]
    per_w = n_pad // NW
    nchunks = per_w // CH

    def body(i, carry):
        base = w * per_w + i * CH
        pltpu.sync_copy(idx_hbm.at[pl.ds(base, CH)], idx_v)
        pltpu.async_copy(tab_hbm.at[idx_v], rows_v, sem).wait()
        pltpu.sync_copy(rows_v, out_hbm.at[pl.ds(base, CH)])
        return carry

    lax.fori_loop(0, nchunks, body, 0)


def _entity_gather(entity_table, ids_pad, n_pad):
    mesh = plsc.VectorSubcoreMesh(core_axis_name="c", subcore_axis_name="s")
    f = functools.partial(
        pl.kernel,
        out_type=jax.ShapeDtypeStruct((n_pad, D), jnp.float32),
        mesh=mesh,
        scratch_types=[
            pltpu.VMEM((CH,), jnp.int32),
            pltpu.VMEM((CH, D), jnp.float32),
            pltpu.SemaphoreType.DMA,
        ],
        compiler_params=pltpu.CompilerParams(needs_layout_passes=False),
    )(_entity_gather_body)
    return f(entity_table, ids_pad)


def _dense_body(raw_ref, q_ref, rel_ref, went_ref, wq_ref, wedge_ref,
                nt_ref, qt_ref, a_ref, c_ref, rp_ref, qp_ref):
    we = wedge_ref[...]
    w1 = we[0 * D:1 * D]
    w2 = we[1 * D:2 * D]
    w3 = we[2 * D:3 * D]
    w4 = we[3 * D:4 * D]
    nt = jnp.dot(raw_ref[...], went_ref[...], preferred_element_type=jnp.float32)
    nt_ref[...] = nt
    a_ref[...] = jnp.dot(nt, w1, preferred_element_type=jnp.float32)
    c_ref[...] = jnp.dot(nt, w3, preferred_element_type=jnp.float32)
    qt = jnp.dot(q_ref[...], wq_ref[...], preferred_element_type=jnp.float32)
    qt_ref[...] = qt
    qp_ref[...] = jnp.dot(qt, w4, preferred_element_type=jnp.float32)
    rp_ref[...] = jnp.dot(rel_ref[...], w2, preferred_element_type=jnp.float32)


def _dense(raw_pad, question_emb, relation_table, W_ent, W_q, W_edge):
    n_pad = raw_pad.shape[0]
    b = question_emb.shape[0]
    n_rel = relation_table.shape[0]
    outs = (
        jax.ShapeDtypeStruct((n_pad, D), jnp.float32),
        jax.ShapeDtypeStruct((b, D), jnp.float32),
        jax.ShapeDtypeStruct((n_pad, D), jnp.float32),
        jax.ShapeDtypeStruct((n_pad, D), jnp.float32),
        jax.ShapeDtypeStruct((n_rel, D), jnp.float32),
        jax.ShapeDtypeStruct((b, D), jnp.float32),
    )
    return pl.pallas_call(_dense_body, out_shape=outs)(
        raw_pad, question_emb, relation_table, W_ent, W_q, W_edge)


def _edge_body(a_hbm, rp_hbm, c_hbm, qp_hbm, h_hbm, t_hbm, r_hbm, b_hbm,
               gid_hbm, out_hbm, hg_hbm, tg_hbm,
               hv, tv, rv, bv, ab, rb, cb, qb, gid_v, hg_v, tg_v, sem):
    w = lax.axis_index("s") * NC + lax.axis_index("c")
    ne_pad = out_hbm.shape[0]
    per_w = ne_pad // NW
    nchunks = per_w // CH

    pltpu.sync_copy(gid_hbm, gid_v)

    def chunk(i, carry):
        base = w * per_w + i * CH
        pltpu.sync_copy(h_hbm.at[pl.ds(base, CH)], hv)
        pltpu.sync_copy(t_hbm.at[pl.ds(base, CH)], tv)
        pltpu.sync_copy(r_hbm.at[pl.ds(base, CH)], rv)
        pltpu.sync_copy(b_hbm.at[pl.ds(base, CH)], bv)
        ca = pltpu.async_copy(a_hbm.at[hv], ab, sem)
        cr = pltpu.async_copy(rp_hbm.at[rv], rb, sem)
        cc = pltpu.async_copy(c_hbm.at[tv], cb, sem)
        cq = pltpu.async_copy(qp_hbm.at[bv], qb, sem)
        ca.wait()
        cr.wait()
        cc.wait()
        cq.wait()

        def edge_e(e, inner):
            for col in range(D // 16):
                s = pl.ds(col * 16, 16)
                ab[e, s] = ab[e, s] + rb[e, s] + cb[e, s] + qb[e, s]
            return inner

        lax.fori_loop(0, CH, edge_e, 0)

        def g16(j, inner):
            s = pl.ds(j * 16, 16)
            hg_v[s] = plsc.load_gather(gid_v, [hv[s]])
            tg_v[s] = plsc.load_gather(gid_v, [tv[s]])
            return inner

        lax.fori_loop(0, CH // 16, g16, 0)

        pltpu.sync_copy(ab, out_hbm.at[pl.ds(base, CH)])
        pltpu.sync_copy(hg_v, hg_hbm.at[pl.ds(base, CH)])
        pltpu.sync_copy(tg_v, tg_hbm.at[pl.ds(base, CH)])
        return carry

    lax.fori_loop(0, nchunks, chunk, 0)


def _edge_stage(A, Rp, C, Qp, heads_pad, tails_pad, rels_pad, batch_pad, gids):
    ne_pad = heads_pad.shape[0]
    mesh = plsc.VectorSubcoreMesh(core_axis_name="c", subcore_axis_name="s")
    n_nodes = gids.shape[0]
    f = functools.partial(
        pl.kernel,
        out_type=(
            jax.ShapeDtypeStruct((ne_pad, D), jnp.float32),
            jax.ShapeDtypeStruct((ne_pad,), jnp.int32),
            jax.ShapeDtypeStruct((ne_pad,), jnp.int32),
        ),
        mesh=mesh,
        scratch_types=[
            pltpu.VMEM((CH,), jnp.int32),
            pltpu.VMEM((CH,), jnp.int32),
            pltpu.VMEM((CH,), jnp.int32),
            pltpu.VMEM((CH,), jnp.int32),
            pltpu.VMEM((CH, D), jnp.float32),
            pltpu.VMEM((CH, D), jnp.float32),
            pltpu.VMEM((CH, D), jnp.float32),
            pltpu.VMEM((CH, D), jnp.float32),
            pltpu.VMEM((n_nodes,), jnp.int32),
            pltpu.VMEM((CH,), jnp.int32),
            pltpu.VMEM((CH,), jnp.int32),
            pltpu.SemaphoreType.DMA,
        ],
        compiler_params=pltpu.CompilerParams(needs_layout_passes=False),
    )(_edge_body)
    return f(A, Rp, C, Qp, heads_pad, tails_pad, rels_pad, batch_pad, gids)


def kernel(question_emb, entity_table, relation_table, W_ent, W_q, W_edge,
           node_embedding_ids, node_global_ids, edge_index, edge_relations,
           edge_batch):
    n_nodes = node_embedding_ids.shape[0]
    n_edges = edge_relations.shape[0]
    n_pad = _round_up(n_nodes, NW * CH)
    ne_pad = _round_up(n_edges, NW * CH)

    ids_pad = jnp.pad(node_embedding_ids, (0, n_pad - n_nodes))
    heads_pad = jnp.pad(edge_index[0], (0, ne_pad - n_edges))
    tails_pad = jnp.pad(edge_index[1], (0, ne_pad - n_edges))
    rels_pad = jnp.pad(edge_relations, (0, ne_pad - n_edges))
    batch_pad = jnp.pad(edge_batch, (0, ne_pad - n_edges))

    raw_pad = _entity_gather(entity_table, ids_pad, n_pad)
    nt_pad, qt, A, C, Rp, Qp = _dense(
        raw_pad, question_emb, relation_table, W_ent, W_q, W_edge)

    edge_tok, hg, tg = _edge_stage(
        A, Rp, C, Qp, heads_pad, tails_pad, rels_pad, batch_pad,
        node_global_ids)

    return (edge_tok[:n_edges], nt_pad[:n_nodes], qt,
            hg[:n_edges], tg[:n_edges])


# speedup vs baseline: 9.1036x; 2.6434x over previous
import functools

import jax
import jax.numpy as jnp
from jax import lax
from jax.experimental import pallas as pl
from jax.experimental.pallas import tpu as pltpu
from jax.experimental.pallas import tpu_sc as plsc

D = 128
NC = 2
NS = 16
NW = NC * NS
L = 16

CHN = 128
NCH_N = 3
CH = 80
NCH_E = 126


def _round_up(x, m):
    return (x + m - 1) // m * m


def _entity_gather_body(tab_hbm, idx_hbm, out_hbm, idxb, r0, r1, r2, sem):
    w = lax.axis_index("s") * NC + lax.axis_index("c")
    pltpu.sync_copy(idx_hbm.at[w], idxb)
    c0 = pltpu.async_copy(tab_hbm.at[idxb.at[0]], r0, sem)
    c1 = pltpu.async_copy(tab_hbm.at[idxb.at[1]], r1, sem)
    c2 = pltpu.async_copy(tab_hbm.at[idxb.at[2]], r2, sem)
    base = w * (NCH_N * CHN)
    c0.wait()
    pltpu.sync_copy(r0, out_hbm.at[pl.ds(base, CHN)])
    c1.wait()
    pltpu.sync_copy(r1, out_hbm.at[pl.ds(base + CHN, CHN)])
    c2.wait()
    pltpu.sync_copy(r2, out_hbm.at[pl.ds(base + 2 * CHN, CHN)])


def _entity_gather(entity_table, ids2d, n_pad):
    mesh = plsc.VectorSubcoreMesh(core_axis_name="c", subcore_axis_name="s")
    f = functools.partial(
        pl.kernel,
        out_type=jax.ShapeDtypeStruct((n_pad, D), jnp.float32),
        mesh=mesh,
        scratch_types=[
            pltpu.VMEM((NCH_N, CHN), jnp.int32),
            pltpu.VMEM((CHN, D), jnp.float32),
            pltpu.VMEM((CHN, D), jnp.float32),
            pltpu.VMEM((CHN, D), jnp.float32),
            pltpu.SemaphoreType.DMA,
        ],
        compiler_params=pltpu.CompilerParams(needs_layout_passes=False),
    )(_entity_gather_body)
    return f(entity_table, ids2d)


def _dense_body(raw_ref, q_ref, rel_ref, went_ref, wq_ref, wedge_ref,
                nt_ref, qt_ref, a_ref, c_ref, rq_ref):
    we = wedge_ref[...]
    w1 = we[0 * D:1 * D]
    w2 = we[1 * D:2 * D]
    w3 = we[2 * D:3 * D]
    w4 = we[3 * D:4 * D]
    nt = jnp.dot(raw_ref[...], went_ref[...], preferred_element_type=jnp.float32)
    nt_ref[...] = nt
    a_ref[...] = jnp.dot(nt, w1, preferred_element_type=jnp.float32)
    c_ref[...] = jnp.dot(nt, w3, preferred_element_type=jnp.float32)
    qt = jnp.dot(q_ref[...], wq_ref[...], preferred_element_type=jnp.float32)
    qt_ref[...] = qt
    qp = jnp.dot(qt, w4, preferred_element_type=jnp.float32)
    rp = jnp.dot(rel_ref[...], w2, preferred_element_type=jnp.float32)
    b = qp.shape[0]
    n_rel = rp.shape[0]
    rq_ref[...] = (qp[:, None, :] + rp[None, :, :]).reshape(b * n_rel, D)


def _dense(raw_pad, question_emb, relation_table, W_ent, W_q, W_edge):
    n_pad = raw_pad.shape[0]
    b = question_emb.shape[0]
    n_rel = relation_table.shape[0]
    outs = (
        jax.ShapeDtypeStruct((n_pad, D), jnp.float32),
        jax.ShapeDtypeStruct((b, D), jnp.float32),
        jax.ShapeDtypeStruct((n_pad, D), jnp.float32),
        jax.ShapeDtypeStruct((n_pad, D), jnp.float32),
        jax.ShapeDtypeStruct((b * n_rel, D), jnp.float32),
    )
    return pl.pallas_call(_dense_body, out_shape=outs)(
        raw_pad, question_emb, relation_table, W_ent, W_q, W_edge)


def _make_edge_body(n_rel):
    def _edge_body(a_hbm, rq_hbm, c_hbm, cidx_hbm, gid_hbm,
                   out_hbm, hg_hbm, tg_hbm,
                   idxb, rqib, ab, rqb, cb, hgb, tgb, gid_v, semg, semo):
        w = lax.axis_index("s") * NC + lax.axis_index("c")
        pltpu.sync_copy(gid_hbm, gid_v)

        def stage(g, t):
            pltpu.sync_copy(cidx_hbm.at[g], idxb.at[t])
            for j in range(CH // L):
                s = pl.ds(j * L, L)
                rqib[t, s] = idxb[t, 3, s] * n_rel + idxb[t, 2, s]
            pltpu.async_copy(a_hbm.at[idxb.at[t, 0]], ab.at[t], semg.at[t])
            pltpu.async_copy(rq_hbm.at[rqib.at[t]], rqb.at[t], semg.at[t])
            pltpu.async_copy(c_hbm.at[idxb.at[t, 1]], cb.at[t], semg.at[t])

        def drain_gathers(t):
            pltpu.make_async_copy(a_hbm.at[idxb.at[t, 0]], ab.at[t],
                                  semg.at[t]).wait()
            pltpu.make_async_copy(rq_hbm.at[rqib.at[t]], rqb.at[t],
                                  semg.at[t]).wait()
            pltpu.make_async_copy(c_hbm.at[idxb.at[t, 1]], cb.at[t],
                                  semg.at[t]).wait()

        def drain_out(t):
            pltpu.make_async_copy(cb.at[t], out_hbm.at[pl.ds(0, CH)],
                                  semo.at[t]).wait()
            pltpu.make_async_copy(hgb.at[t], hg_hbm.at[pl.ds(0, CH)],
                                  semo.at[t]).wait()
            pltpu.make_async_copy(tgb.at[t], tg_hbm.at[pl.ds(0, CH)],
                                  semo.at[t]).wait()

        def compute(t):
            def edge_e(e, carry):
                for col in range(D // L):
                    s = pl.ds(col * L, L)
                    cb[t, e, s] = ab[t, e, s] + rqb[t, e, s] + cb[t, e, s]
                return carry
            lax.fori_loop(0, CH, edge_e, 0)
            for j in range(CH // L):
                s = pl.ds(j * L, L)
                hgb[t, s] = plsc.load_gather(gid_v, [idxb[t, 0, s]])
                tgb[t, s] = plsc.load_gather(gid_v, [idxb[t, 1, s]])

        def issue_out(g, t):
            base = g * CH
            pltpu.async_copy(cb.at[t], out_hbm.at[pl.ds(base, CH)], semo.at[t])
            pltpu.async_copy(hgb.at[t], hg_hbm.at[pl.ds(base, CH)], semo.at[t])
            pltpu.async_copy(tgb.at[t], tg_hbm.at[pl.ds(base, CH)], semo.at[t])

        g0 = w * NCH_E
        stage(g0, 0)

        def body(i, carry):
            for t in range(3):
                j = 3 * i + t
                g = g0 + j
                nxt = (t + 1) % 3
                drain_gathers(t)
                if t == 2:
                    drain_out(nxt)
                else:
                    @pl.when(i >= 1)
                    def _():
                        drain_out(nxt)
                stage(g + 1, nxt)
                compute(t)
                issue_out(g, t)
            return carry

        lax.fori_loop(0, NCH_E // 3, body, 0)

        drain_gathers(0)
        drain_out(1)
        drain_out(2)

    return _edge_body


def _edge_stage(A, RQ, C, cidx, gids, n_rel, ne_pad):
    mesh = plsc.VectorSubcoreMesh(core_axis_name="c", subcore_axis_name="s")
    n_nodes = gids.shape[0]
    f = functools.partial(
        pl.kernel,
        out_type=(
            jax.ShapeDtypeStruct((ne_pad, D), jnp.float32),
            jax.ShapeDtypeStruct((ne_pad,), jnp.int32),
            jax.ShapeDtypeStruct((ne_pad,), jnp.int32),
        ),
        mesh=mesh,
        scratch_types=[
            pltpu.VMEM((3, 4, CH), jnp.int32),
            pltpu.VMEM((3, CH), jnp.int32),
            pltpu.VMEM((3, CH, D), jnp.float32),
            pltpu.VMEM((3, CH, D), jnp.float32),
            pltpu.VMEM((3, CH, D), jnp.float32),
            pltpu.VMEM((3, CH), jnp.int32),
            pltpu.VMEM((3, CH), jnp.int32),
            pltpu.VMEM((n_nodes,), jnp.int32),
            pltpu.SemaphoreType.DMA((3,)),
            pltpu.SemaphoreType.DMA((3,)),
        ],
        compiler_params=pltpu.CompilerParams(needs_layout_passes=False),
    )(_make_edge_body(n_rel))
    return f(A, RQ, C, cidx, gids)


def kernel(question_emb, entity_table, relation_table, W_ent, W_q, W_edge,
           node_embedding_ids, node_global_ids, edge_index, edge_relations,
           edge_batch):
    n_nodes = node_embedding_ids.shape[0]
    n_edges = edge_relations.shape[0]
    n_rel = relation_table.shape[0]
    n_pad = _round_up(n_nodes, NW * NCH_N * CHN)
    ne_pad = NW * NCH_E * CH
    assert ne_pad >= n_edges

    ids2d = jnp.pad(node_embedding_ids,
                    (0, n_pad - n_nodes)).reshape(NW, NCH_N, CHN)

    pad_len = ne_pad + CH - n_edges
    hp = jnp.pad(edge_index[0], (0, pad_len))
    tp = jnp.pad(edge_index[1], (0, pad_len))
    rp = jnp.pad(edge_relations, (0, pad_len))
    bp = jnp.pad(edge_batch, (0, pad_len))
    cidx = (jnp.stack([hp, tp, rp, bp], 0)
            .reshape(4, ne_pad // CH + 1, CH)
            .transpose(1, 0, 2))

    raw_pad = _entity_gather(entity_table, ids2d, n_pad)
    nt_pad, qt, A, C, RQ = _dense(
        raw_pad, question_emb, relation_table, W_ent, W_q, W_edge)

    edge_tok, hg, tg = _edge_stage(A, RQ, C, cidx, node_global_ids,
                                   n_rel, ne_pad)

    return (edge_tok[:n_edges], nt_pad[:n_nodes], qt,
            hg[:n_edges], tg[:n_edges])
